# baseline (speedup 1.0000x reference)
import functools

import jax
import jax.numpy as jnp
from jax import lax
from jax.experimental import pallas as pl
from jax.experimental.pallas import tpu as pltpu
from jax.experimental.pallas import tpu_sc as plsc

B = 4096
L = 200
HIDDEN = 64
N_ROWS = B * L
NW = 32
ROWS_PER_W = N_ROWS // NW
CHUNK = 256
N_CHUNKS = ROWS_PER_W // CHUNK
SUB = 128
NSUB = CHUNK // SUB


def _rsqrt(x):
    i = lax.bitcast_convert_type(x, jnp.int32)
    i = jnp.int32(0x5F3759DF) - lax.shift_right_arithmetic(i, 1)
    y = lax.bitcast_convert_type(i, jnp.float32)
    for _ in range(3):
        y = y * (1.5 - 0.5 * x * y * y)
    return y


def _body(ids_hbm, table_hbm, pos_hbm, gamma_hbm, beta_hbm, out_hbm,
          pos_v, gb_v, idx_v, rows_v, gsem):
    cid = lax.axis_index("c")
    sid = lax.axis_index("s")
    wid = sid * 2 + cid
    base = wid * ROWS_PER_W

    pltpu.sync_copy(pos_hbm.at[pl.ds(0, L), :], pos_v)
    pltpu.sync_copy(gamma_hbm, gb_v.at[0])
    pltpu.sync_copy(beta_hbm, gb_v.at[1])

    gamma = [gb_v[0, pl.ds(16 * h, 16)] for h in range(4)]
    beta = [gb_v[1, pl.ds(16 * h, 16)] for h in range(4)]

    def chunk_body(c, carry):
        g0 = base + c * CHUNK
        for j in range(NSUB):
            pltpu.sync_copy(ids_hbm.at[pl.ds(g0 + j * SUB, SUB)],
                            idx_v.at[j])
        for j in range(NSUB):
            pltpu.async_copy(table_hbm.at[idx_v.at[j]],
                             rows_v.at[pl.ds(j * SUB, SUB), :], gsem).wait()

        p0 = lax.rem(c * CHUNK, jnp.int32(L))

        def row_body(r, carry2):
            p = lax.rem(p0 + r, jnp.int32(L))
            x = [rows_v[r, pl.ds(16 * h, 16)] + pos_v[p, pl.ds(16 * h, 16)]
                 for h in range(4)]
            s = (x[0] + x[1]) + (x[2] + x[3])
            q = (x[0] * x[0] + x[1] * x[1]) + (x[2] * x[2] + x[3] * x[3])
            mean = jnp.sum(s) * (1.0 / HIDDEN)
            var = jnp.sum(q) * (1.0 / HIDDEN) - mean * mean
            inv = _rsqrt(var + 1e-5)
            for h in range(4):
                g = gamma[h] * inv
                rows_v[r, pl.ds(16 * h, 16)] = (x[h] - mean) * g + beta[h]
            return carry2

        lax.fori_loop(0, CHUNK, row_body, 0, unroll=2)
        pltpu.sync_copy(rows_v, out_hbm.at[pl.ds(g0, CHUNK), :])
        return carry

    lax.fori_loop(0, N_CHUNKS, chunk_body, 0)


def kernel(input_ids, token_table, pos_table, gamma, beta):
    ids_flat = input_ids.reshape(-1).astype(jnp.int32)
    mesh = plsc.VectorSubcoreMesh(core_axis_name="c", subcore_axis_name="s")
    run = pl.kernel(
        _body,
        out_type=jax.ShapeDtypeStruct((N_ROWS, HIDDEN), jnp.float32),
        mesh=mesh,
        compiler_params=pltpu.CompilerParams(needs_layout_passes=False,
                                             use_tc_tiling_on_sc=False),
        scratch_types=[
            pltpu.VMEM((L, HIDDEN), jnp.float32),
            pltpu.VMEM((2, HIDDEN), jnp.float32),
            pltpu.VMEM((NSUB, SUB), jnp.int32),
            pltpu.VMEM((CHUNK, HIDDEN), jnp.float32),
            pltpu.SemaphoreType.DMA,
        ],
    )
    out = run(ids_flat, token_table, pos_table, gamma, beta)
    return out.reshape(B, L, HIDDEN)


# speedup vs baseline: 1.7184x; 1.7184x over previous
import jax
import jax.numpy as jnp
from jax import lax
from jax.experimental import pallas as pl
from jax.experimental.pallas import tpu as pltpu
from jax.experimental.pallas import tpu_sc as plsc

B = 4096
L = 200
HIDDEN = 64
N_ROWS = B * L
NW = 32
ROWS_PER_W = N_ROWS // NW
CHUNK = 256
N_CHUNKS = ROWS_PER_W // CHUNK
SUB = 128
NSUB = CHUNK // SUB


def _rsqrt(x):
    i = lax.bitcast_convert_type(x, jnp.int32)
    i = jnp.int32(0x5F3759DF) - lax.shift_right_arithmetic(i, 1)
    y = lax.bitcast_convert_type(i, jnp.float32)
    for _ in range(3):
        y = y * (1.5 - 0.5 * x * y * y)
    return y


def _body(ids_hbm, table_hbm, pos_hbm, gamma_hbm, beta_hbm, out_hbm,
          pos_v, gb_v, idx_v, rows0, rows1, gsem0, gsem1, osem0, osem1):
    cid = lax.axis_index("c")
    sid = lax.axis_index("s")
    wid = sid * 2 + cid
    base = wid * ROWS_PER_W
    rows = (rows0, rows1)
    gsem = (gsem0, gsem1)
    osem = (osem0, osem1)

    pltpu.sync_copy(pos_hbm.at[pl.ds(0, L), :], pos_v)
    pltpu.sync_copy(gamma_hbm, gb_v.at[0])
    pltpu.sync_copy(beta_hbm, gb_v.at[1])

    gamma = [gb_v[0, pl.ds(16 * h, 16)] for h in range(4)]
    beta = [gb_v[1, pl.ds(16 * h, 16)] for h in range(4)]

    def fire_gather(c, b):
        g0 = base + c * CHUNK
        for j in range(NSUB):
            pltpu.sync_copy(ids_hbm.at[pl.ds(g0 + j * SUB, SUB)],
                            idx_v.at[b, j])
        for j in range(NSUB):
            pltpu.async_copy(table_hbm.at[idx_v.at[b, j]],
                             rows[b].at[pl.ds(j * SUB, SUB), :], gsem[b])

    def process(c, b):
        g0 = base + c * CHUNK
        for j in range(NSUB):
            pltpu.make_async_copy(table_hbm.at[idx_v.at[b, j]],
                                  rows[b].at[pl.ds(j * SUB, SUB), :],
                                  gsem[b]).wait()
        @pl.when(c >= 2)
        def _():
            pltpu.make_async_copy(
                rows[b], out_hbm.at[pl.ds(base, CHUNK), :], osem[b]).wait()

        p0 = lax.rem(jnp.int32(c * CHUNK), jnp.int32(L))

        @plsc.parallel_loop(0, CHUNK, 1, unroll=8)
        def _(r):
            p = lax.rem(p0 + r, jnp.int32(L))
            x = [rows[b][r, pl.ds(16 * h, 16)] + pos_v[p, pl.ds(16 * h, 16)]
                 for h in range(4)]
            s = (x[0] + x[1]) + (x[2] + x[3])
            q = (x[0] * x[0] + x[1] * x[1]) + (x[2] * x[2] + x[3] * x[3])
            mean = jnp.sum(s) * (1.0 / HIDDEN)
            var = jnp.sum(q) * (1.0 / HIDDEN) - mean * mean
            inv = _rsqrt(var + 1e-5)
            for h in range(4):
                g = gamma[h] * inv
                rows[b][r, pl.ds(16 * h, 16)] = (x[h] - mean) * g + beta[h]

        pltpu.async_copy(rows[b], out_hbm.at[pl.ds(g0, CHUNK), :], osem[b])

    fire_gather(0, 0)

    def pair_body(g, carry):
        for bb in range(2):
            c = g * 2 + bb

            @pl.when(c + 1 < N_CHUNKS)
            def _():
                fire_gather(c + 1, 1 - bb)

            process(c, bb)
        return carry

    lax.fori_loop(0, N_CHUNKS // 2, pair_body, 0)

    for bb in range(2):
        pltpu.make_async_copy(
            rows[bb], out_hbm.at[pl.ds(base, CHUNK), :], osem[bb]).wait()


def kernel(input_ids, token_table, pos_table, gamma, beta):
    ids_flat = input_ids.reshape(-1).astype(jnp.int32)
    mesh = plsc.VectorSubcoreMesh(core_axis_name="c", subcore_axis_name="s")
    run = pl.kernel(
        _body,
        out_type=jax.ShapeDtypeStruct((N_ROWS, HIDDEN), jnp.float32),
        mesh=mesh,
        compiler_params=pltpu.CompilerParams(needs_layout_passes=False,
                                             use_tc_tiling_on_sc=False),
        scratch_types=[
            pltpu.VMEM((L, HIDDEN), jnp.float32),
            pltpu.VMEM((2, HIDDEN), jnp.float32),
            pltpu.VMEM((2, NSUB, SUB), jnp.int32),
            pltpu.VMEM((CHUNK, HIDDEN), jnp.float32),
            pltpu.VMEM((CHUNK, HIDDEN), jnp.float32),
            pltpu.SemaphoreType.DMA,
            pltpu.SemaphoreType.DMA,
            pltpu.SemaphoreType.DMA,
            pltpu.SemaphoreType.DMA,
        ],
    )
    out = run(ids_flat, token_table, pos_table, gamma, beta)
    return out.reshape(B, L, HIDDEN)
